# baseline (speedup 1.0000x reference)
import functools

import jax
import jax.numpy as jnp
from jax import lax
from jax.experimental import pallas as pl
from jax.experimental.pallas import tpu as pltpu
from jax.experimental.pallas import tpu_sc as plsc

VOCAB = 1000
B, TT = 1024, 200
NVT = 125
QCOL = 256
PB = 129
Q3_VT = NVT - 3 * 32


def _transpose_gather(t0, t1, t2, t3, idx_hbm, out_hbm,
                      idx_v, buf0, buf1, tb0, tb1, g0, g1, w0, w1):
    wid = lax.axis_index("s") * 2 + lax.axis_index("c")
    q = lax.rem(wid, 4)
    bt = wid // 4
    col0 = pl.multiple_of(128 * bt, 128)
    pltpu.sync_copy(idx_hbm.at[:, pl.ds(col0, 128)], idx_v)

    tables = (t0, t1, t2, t3)

    def start_gather(t_, slot_buf, slot_sem):
        for qq in range(4):
            @pl.when(q == qq)
            def _():
                pltpu.async_copy(tables[qq].at[idx_v.at[t_]], slot_buf, slot_sem)

    def wait_gather(slot_buf, slot_sem):
        pltpu.make_async_copy(t0.at[pl.ds(0, 128)], slot_buf, slot_sem).wait()

    def write_src(tb, nvt):
        return tb.at[pl.ds(0, nvt), :, :, pl.ds(0, 128)]

    def wait_write(tb, sem, nvt):
        pltpu.make_async_copy(
            write_src(tb, nvt),
            out_hbm.at[0, pl.ds(0, nvt), pl.ds(0, 1)], sem).wait()

    lane = lax.iota(jnp.int32, 16)
    zerov = jnp.zeros((16,), jnp.int32)
    vt_vecs = [(16 * m + lane) // 8 for m in range(8)]
    s_vecs = [lax.rem(16 * m + lane, 8) for m in range(8)]

    def assemble(t_, buf):
        for b_ in range(2):
            tb = (tb0, tb1)[b_]
            sem = (w0, w1)[b_]

            @pl.when(t_ >= 1)
            def _():
                @pl.when((q < 3) | (b_ == 0))
                def _():
                    wait_write(tb, sem, 16)
                if b_ == 1:
                    @pl.when(q == 3)
                    def _():
                        wait_write(tb, sem, Q3_VT - 16)

            def row_body(r, carry):
                bv = jnp.full((16,), r, jnp.int32)
                for m in range(8):
                    x = buf[r, pl.ds(128 * b_ + 16 * m, 16)]
                    plsc.store_scatter(tb, [vt_vecs[m], zerov, s_vecs[m], bv], x)
                return carry

            lax.fori_loop(0, 128, row_body, 0)
            vt0 = q * 32 + 16 * b_

            @pl.when((q < 3) | (b_ == 0))
            def _():
                pltpu.async_copy(
                    write_src(tb, 16),
                    out_hbm.at[t_, pl.ds(vt0, 16), pl.ds(bt, 1)], sem)
            if b_ == 1:
                @pl.when(q == 3)
                def _():
                    pltpu.async_copy(
                        write_src(tb, Q3_VT - 16),
                        out_hbm.at[t_, pl.ds(vt0, Q3_VT - 16), pl.ds(bt, 1)],
                        sem)

    start_gather(0, buf0, g0)

    def ring(h, carry):
        te = 2 * h
        start_gather(te + 1, buf1, g1)
        wait_gather(buf0, g0)
        assemble(te, buf0)

        @pl.when(te + 2 < TT)
        def _():
            start_gather(te + 2, buf0, g0)
        wait_gather(buf1, g1)
        assemble(te + 1, buf1)
        return carry

    lax.fori_loop(0, TT // 2, ring, 0)
    wait_write(tb0, w0, 16)

    @pl.when(q < 3)
    def _():
        wait_write(tb1, w1, 16)

    @pl.when(q == 3)
    def _():
        wait_write(tb1, w1, Q3_VT - 16)


def kernel(idx, embed_weight):
    idx_t = idx.astype(jnp.int32).T.reshape(TT, B)
    quarters = [embed_weight[:, i * QCOL:(i + 1) * QCOL] for i in range(3)]
    quarters.append(jnp.pad(embed_weight[:, 3 * QCOL:VOCAB],
                            ((0, 0), (0, 4 * QCOL - VOCAB))))

    mesh = plsc.VectorSubcoreMesh(core_axis_name="c", subcore_axis_name="s")
    k = functools.partial(
        pl.kernel,
        out_type=jax.ShapeDtypeStruct((TT, NVT, 8, 8, 128), jnp.float32),
        mesh=mesh,
        compiler_params=pltpu.CompilerParams(needs_layout_passes=False,
                                             use_tc_tiling_on_sc=False),
        scratch_types=[
            pltpu.VMEM((TT, 128), jnp.int32),
            pltpu.VMEM((128, QCOL), jnp.float32),
            pltpu.VMEM((128, QCOL), jnp.float32),
            pltpu.VMEM((16, 1, 8, PB), jnp.float32),
            pltpu.VMEM((16, 1, 8, PB), jnp.float32),
            pltpu.SemaphoreType.DMA,
            pltpu.SemaphoreType.DMA,
            pltpu.SemaphoreType.DMA,
            pltpu.SemaphoreType.DMA,
        ],
    )(_transpose_gather)
    x = k(*quarters, idx_t)
    return x.transpose(2, 4, 0, 1, 3).reshape(B, TT, VOCAB)


# speedup vs baseline: 1.3808x; 1.3808x over previous
import functools

import jax
import jax.numpy as jnp
from jax import lax
from jax.experimental import pallas as pl
from jax.experimental.pallas import tpu as pltpu
from jax.experimental.pallas import tpu_sc as plsc

VOCAB = 1000
BODY = 896
TAIL = VOCAB - BODY
NUM_WORKERS = 32
CHUNK = 40


def _gather_rows(body_hbm, tail_hbm, idx_hbm, out_hbm,
                 idx_v, buf0, buf1, tbuf0, tbuf1,
                 gb0, gb1, gt0, gt1, w0, w1):
    per_w = idx_v.shape[0]
    n_chunks = per_w // CHUNK
    wid = lax.axis_index("s") * 2 + lax.axis_index("c")
    base = wid * per_w
    pltpu.sync_copy(idx_hbm.at[pl.ds(base, per_w)], idx_v)

    bufs = (buf0, buf1)
    tbufs = (tbuf0, tbuf1)
    gbs = (gb0, gb1)
    gts = (gt0, gt1)
    ws = (w0, w1)

    lane = lax.iota(jnp.int32, 16)
    last_lanes = BODY + 6 * 16 + lane
    last_mask = lane < (TAIL - 6 * 16)
    last_idx = jnp.where(last_mask, last_lanes, VOCAB - 1)

    def start_gathers(g, slot):
        off = pl.multiple_of(g * CHUNK, CHUNK)
        idx_c = idx_v.at[pl.ds(off, CHUNK)]
        pltpu.async_copy(body_hbm.at[idx_c],
                         bufs[slot].at[:, pl.ds(0, BODY)], gbs[slot])
        pltpu.async_copy(tail_hbm.at[idx_c], tbufs[slot], gts[slot])

    def wait_gathers(slot):
        pltpu.make_async_copy(body_hbm.at[pl.ds(0, CHUNK)],
                              bufs[slot].at[:, pl.ds(0, BODY)],
                              gbs[slot]).wait()
        pltpu.make_async_copy(tail_hbm.at[pl.ds(0, CHUNK)],
                              tbufs[slot], gts[slot]).wait()

    def wait_write(slot):
        pltpu.make_async_copy(bufs[slot], out_hbm.at[pl.ds(0, CHUNK)],
                              ws[slot]).wait()

    def splice(slot):
        buf, tbuf = bufs[slot], tbufs[slot]

        def splice_row(r, carry):
            for k in range(6):
                buf[r, pl.ds(BODY + 16 * k, 16)] = tbuf[r, pl.ds(16 * k, 16)]
            x = tbuf[r, pl.ds(96, 16)]
            plsc.store_scatter(buf, [jnp.full((16,), r, jnp.int32), last_idx],
                               x, mask=last_mask)
            return carry

        lax.fori_loop(0, CHUNK, splice_row, 0)

    def process(g, slot):
        wait_gathers(slot)

        @pl.when(g >= 2)
        def _():
            wait_write(slot)
        splice(slot)
        off = pl.multiple_of(g * CHUNK, CHUNK)
        pltpu.async_copy(bufs[slot], out_hbm.at[pl.ds(base + off, CHUNK)],
                         ws[slot])

    start_gathers(0, 0)

    def ring(h, carry):
        g = 2 * h
        start_gathers(g + 1, 1)
        process(g, 0)

        @pl.when(g + 2 < n_chunks)
        def _():
            start_gathers(g + 2, 0)
        process(g + 1, 1)
        return carry

    lax.fori_loop(0, n_chunks // 2, ring, 0)
    wait_write(0)
    wait_write(1)


def kernel(idx, embed_weight):
    B, T = idx.shape
    N = B * T
    idx_flat = idx.reshape(N).astype(jnp.int32)
    body = embed_weight[:, :BODY]
    tail = jnp.pad(embed_weight[:, BODY:VOCAB], ((0, 0), (0, 128 - TAIL)))
    per_w = N // NUM_WORKERS

    mesh = plsc.VectorSubcoreMesh(core_axis_name="c", subcore_axis_name="s")
    k = functools.partial(
        pl.kernel,
        out_type=jax.ShapeDtypeStruct((N, VOCAB), jnp.float32),
        mesh=mesh,
        compiler_params=pltpu.CompilerParams(needs_layout_passes=False),
        scratch_types=[
            pltpu.VMEM((per_w,), jnp.int32),
            pltpu.VMEM((CHUNK, VOCAB), jnp.float32),
            pltpu.VMEM((CHUNK, VOCAB), jnp.float32),
            pltpu.VMEM((CHUNK, 128), jnp.float32),
            pltpu.VMEM((CHUNK, 128), jnp.float32),
            pltpu.SemaphoreType.DMA,
            pltpu.SemaphoreType.DMA,
            pltpu.SemaphoreType.DMA,
            pltpu.SemaphoreType.DMA,
            pltpu.SemaphoreType.DMA,
            pltpu.SemaphoreType.DMA,
        ],
    )(_gather_rows)
    out = k(body, tail, idx_flat)
    return out.reshape(B, T, VOCAB)
